# baseline (speedup 1.0000x reference)
import functools

import jax
import jax.numpy as jnp
from jax import lax
from jax.experimental import pallas as pl
from jax.experimental.pallas import tpu as pltpu
from jax.experimental.pallas import tpu_sc as plsc

N = 10000
E = 320000
H = 128
G = 64

NC = 2
NS = 16
NW = NC * NS
EPW = E // NW
CHUNK = 128
NCHUNK = 80
EPT = NCHUNK * CHUNK
NPAD = 10240
RPT = NPAD // NS
DUMMY_DST = NPAD - 2
L = 16


def _sc_aggregate(x, packed, zeros):
    mesh = plsc.VectorSubcoreMesh(core_axis_name="c", subcore_axis_name="s")

    @functools.partial(
        pl.kernel,
        out_type=jax.ShapeDtypeStruct((NC, NPAD, H), jnp.float32),
        mesh=mesh,
        scratch_types=[
            pltpu.VMEM((NCHUNK, CHUNK), jnp.int32),
            pltpu.VMEM((1, CHUNK), jnp.int32),
            pltpu.VMEM((1, CHUNK), jnp.int32),
            pltpu.VMEM((1, CHUNK), jnp.int32),
            pltpu.VMEM((1, CHUNK), jnp.int32),
            pltpu.VMEM((CHUNK, H), jnp.float32),
            pltpu.VMEM((CHUNK, H), jnp.float32),
            pltpu.VMEM_SHARED((NPAD, H), jnp.float32),
            pltpu.SemaphoreType.DMA,
            pltpu.SemaphoreType.DMA,
        ],
    )
    def agg(x_hbm, packed_hbm, z_hbm, out_hbm,
            idx_v, sa, da, sb, db, rows_a, rows_b, acc_sh, sem_a, sem_b):
        c = lax.axis_index("c")
        s = lax.axis_index("s")
        wid = c * NS + s
        pltpu.sync_copy(packed_hbm.at[wid], idx_v)
        pltpu.sync_copy(z_hbm.at[pl.ds(s * RPT, RPT)],
                        acc_sh.at[pl.ds(s * RPT, RPT)])
        plsc.subcore_barrier()

        def unpack(j, s_u, d_u):
            for k in range(CHUNK // L):
                w = idx_v[j, pl.ds(k * L, L)]
                s_u[0, pl.ds(k * L, L)] = w >> 16
                d_u[0, pl.ds(k * L, L)] = w & 0xFFFF

        def gather(s_u, rows, sem):
            pltpu.async_copy(x_hbm.at[s_u.at[0]], rows, sem)

        def scat(d_u, rows, sem):
            pltpu.make_async_copy(x_hbm.at[d_u.at[0]], rows, sem).wait()
            pltpu.sync_copy(rows, acc_sh.at[d_u.at[0]], add=True)

        unpack(0, sa, da)
        gather(sa, rows_a, sem_a)

        def pair(j, issue_next):
            unpack(j + 1, sb, db)
            gather(sb, rows_b, sem_b)
            scat(da, rows_a, sem_a)
            if issue_next:
                unpack(j + 2, sa, da)
                gather(sa, rows_a, sem_a)
            scat(db, rows_b, sem_b)

        def body(p, carry):
            pair(2 * p, True)
            return carry

        assert NCHUNK % 2 == 0
        lax.fori_loop(0, NCHUNK // 2 - 1, body, 0)
        pair(NCHUNK - 2, False)
        plsc.subcore_barrier()
        pltpu.sync_copy(acc_sh.at[pl.ds(s * RPT, RPT)],
                        out_hbm.at[c, pl.ds(s * RPT, RPT)])

    return agg(x, packed, zeros)


_BLK = 1000


def _tc_mid(p, x, W_rel1, b_rel1, W_root1):
    def body(p_ref, x_ref, wr_ref, b_ref, wt_ref, o_ref):
        a = p_ref[0] + p_ref[1]
        h = (jnp.dot(a, wr_ref[...], preferred_element_type=jnp.float32)
             + b_ref[...]
             + jnp.dot(x_ref[...], wt_ref[...],
                       preferred_element_type=jnp.float32))
        o_ref[...] = jnp.maximum(h, 0.0)

    return pl.pallas_call(
        body,
        grid=(N // _BLK,),
        in_specs=[
            pl.BlockSpec((NC, _BLK, H), lambda i: (0, i, 0)),
            pl.BlockSpec((_BLK, H), lambda i: (i, 0)),
            pl.BlockSpec((H, H), lambda i: (0, 0)),
            pl.BlockSpec((1, H), lambda i: (0, 0)),
            pl.BlockSpec((H, H), lambda i: (0, 0)),
        ],
        out_specs=pl.BlockSpec((_BLK, H), lambda i: (i, 0)),
        out_shape=jax.ShapeDtypeStruct((N, H), jnp.float32),
    )(p, x, W_rel1, b_rel1, W_root1)


def _tc_pool(p, h, W_rel2, b_rel2, W_root2, batch3):
    nblk = N // _BLK

    def body(p_ref, h_ref, wr_ref, b_ref, wt_ref, bt_ref, o_ref, acc, cnt):
        i = pl.program_id(0)
        a = p_ref[0] + p_ref[1]
        h2 = (jnp.dot(a, wr_ref[...], preferred_element_type=jnp.float32)
              + b_ref[...]
              + jnp.dot(h_ref[...], wt_ref[...],
                        preferred_element_type=jnp.float32))
        seg = bt_ref[0]
        gids = lax.broadcasted_iota(jnp.int32, (G, _BLK), 0)
        mask = (seg == gids).astype(jnp.float32)

        @pl.when(i == 0)
        def _():
            acc[...] = jnp.zeros_like(acc)
            cnt[...] = jnp.zeros_like(cnt)

        acc[...] += jnp.dot(mask, h2, preferred_element_type=jnp.float32)
        cnt[...] += jnp.broadcast_to(
            jnp.sum(mask, axis=1, keepdims=True), (G, H))

        @pl.when(i == nblk - 1)
        def _():
            o_ref[...] = jnp.maximum(
                acc[...] / jnp.maximum(cnt[...], 1.0), 0.0)

    return pl.pallas_call(
        body,
        grid=(nblk,),
        in_specs=[
            pl.BlockSpec((NC, _BLK, H), lambda i: (0, i, 0)),
            pl.BlockSpec((_BLK, H), lambda i: (i, 0)),
            pl.BlockSpec((H, H), lambda i: (0, 0)),
            pl.BlockSpec((1, H), lambda i: (0, 0)),
            pl.BlockSpec((H, H), lambda i: (0, 0)),
            pl.BlockSpec((1, 1, _BLK), lambda i: (i, 0, 0)),
        ],
        out_specs=pl.BlockSpec((G, H), lambda i: (0, 0)),
        out_shape=jax.ShapeDtypeStruct((G, H), jnp.float32),
        scratch_shapes=[
            pltpu.VMEM((G, H), jnp.float32),
            pltpu.VMEM((G, H), jnp.float32),
        ],
    )(p, h, W_rel2, b_rel2, W_root2, batch3)


def kernel(x, edge_index, batch, W_rel1, b_rel1, W_root1,
           W_rel2, b_rel2, W_root2):
    packed = (edge_index[0] << 16) | edge_index[1]
    packed = packed.reshape(NW, EPW)
    packed = jnp.pad(packed, ((0, 0), (0, EPT - EPW)),
                     constant_values=DUMMY_DST)
    packed = packed.reshape(NW, NCHUNK, CHUNK)
    zeros = jnp.zeros((NPAD, H), jnp.float32)
    batch3 = batch.reshape(N // _BLK, 1, _BLK)

    p1 = _sc_aggregate(x, packed, zeros)
    hmid = _tc_mid(p1, x, W_rel1, b_rel1.reshape(1, H), W_root1)
    p2 = _sc_aggregate(hmid, packed, zeros)
    return _tc_pool(p2, hmid, W_rel2, b_rel2.reshape(1, H),
                    W_root2, batch3)


# speedup vs baseline: 2.8422x; 2.8422x over previous
import functools

import jax
import jax.numpy as jnp
from jax import lax
from jax.experimental import pallas as pl
from jax.experimental.pallas import tpu as pltpu
from jax.experimental.pallas import tpu_sc as plsc

N = 10000
E = 320000
H = 128
G = 64

NC = 2
NS = 16
NW = NC * NS
EPW = E // NW
CHUNK = 80
NCHUNK = EPW // CHUNK
NPAD = 10240
RPT = NPAD // NS


def _sc_aggregate(x, src_flat, dst, zeros):
    mesh = plsc.VectorSubcoreMesh(core_axis_name="c", subcore_axis_name="s")

    @functools.partial(
        pl.kernel,
        out_type=jax.ShapeDtypeStruct((NC, NPAD, H), jnp.float32),
        mesh=mesh,
        scratch_types=[
            pltpu.VMEM((EPW,), jnp.int32),
            pltpu.VMEM((NCHUNK, CHUNK), jnp.int32),
            pltpu.VMEM((CHUNK, H), jnp.float32),
            pltpu.VMEM((CHUNK, H), jnp.float32),
            pltpu.VMEM_SHARED((NPAD, H), jnp.float32),
            pltpu.SemaphoreType.DMA,
            pltpu.SemaphoreType.DMA,
        ],
    )
    def agg(x_hbm, src_hbm, dst_hbm, z_hbm, out_hbm,
            src_v, dst_v, rows_a, rows_b, acc_sh, sem_a, sem_b):
        c = lax.axis_index("c")
        s = lax.axis_index("s")
        wid = c * NS + s
        pltpu.sync_copy(src_hbm.at[pl.ds(wid * EPW, EPW)], src_v)
        pltpu.sync_copy(dst_hbm.at[wid], dst_v)
        pltpu.sync_copy(z_hbm.at[pl.ds(s * RPT, RPT)],
                        acc_sh.at[pl.ds(s * RPT, RPT)])
        plsc.subcore_barrier()

        def gather(j, rows, sem):
            pltpu.async_copy(
                x_hbm.at[src_v.at[pl.ds(j * CHUNK, CHUNK)]], rows, sem)

        def scat(j, rows, sem):
            pltpu.make_async_copy(
                x_hbm.at[src_v.at[pl.ds(j * CHUNK, CHUNK)]],
                rows, sem).wait()
            pltpu.sync_copy(rows, acc_sh.at[dst_v.at[j]], add=True)

        gather(0, rows_a, sem_a)

        def pair(j, issue_next):
            gather(j + 1, rows_b, sem_b)
            scat(j, rows_a, sem_a)
            if issue_next:
                gather(j + 2, rows_a, sem_a)
            scat(j + 1, rows_b, sem_b)

        def body(p, carry):
            pair(2 * p, True)
            return carry

        lax.fori_loop(0, NCHUNK // 2, body, 0)
        scat(NCHUNK - 1, rows_a, sem_a)
        plsc.subcore_barrier()
        pltpu.sync_copy(acc_sh.at[pl.ds(s * RPT, RPT)],
                        out_hbm.at[c, pl.ds(s * RPT, RPT)])

    return agg(x, src_flat, dst, zeros)


_BLK = 1000


def _tc_mid(p, x, W_rel1, b_rel1, W_root1):
    def body(p_ref, x_ref, wr_ref, b_ref, wt_ref, o_ref):
        a = p_ref[0] + p_ref[1]
        h = (jnp.dot(a, wr_ref[...], preferred_element_type=jnp.float32)
             + b_ref[...]
             + jnp.dot(x_ref[...], wt_ref[...],
                       preferred_element_type=jnp.float32))
        o_ref[...] = jnp.maximum(h, 0.0)

    return pl.pallas_call(
        body,
        grid=(N // _BLK,),
        in_specs=[
            pl.BlockSpec((NC, _BLK, H), lambda i: (0, i, 0)),
            pl.BlockSpec((_BLK, H), lambda i: (i, 0)),
            pl.BlockSpec((H, H), lambda i: (0, 0)),
            pl.BlockSpec((1, H), lambda i: (0, 0)),
            pl.BlockSpec((H, H), lambda i: (0, 0)),
        ],
        out_specs=pl.BlockSpec((_BLK, H), lambda i: (i, 0)),
        out_shape=jax.ShapeDtypeStruct((N, H), jnp.float32),
    )(p, x, W_rel1, b_rel1, W_root1)


def _tc_pool(p, h, W_rel2, b_rel2, W_root2, batch3):
    nblk = N // _BLK

    def body(p_ref, h_ref, wr_ref, b_ref, wt_ref, bt_ref, o_ref, acc, cnt):
        i = pl.program_id(0)
        a = p_ref[0] + p_ref[1]
        h2 = (jnp.dot(a, wr_ref[...], preferred_element_type=jnp.float32)
              + b_ref[...]
              + jnp.dot(h_ref[...], wt_ref[...],
                        preferred_element_type=jnp.float32))
        seg = bt_ref[0]
        gids = lax.broadcasted_iota(jnp.int32, (G, _BLK), 0)
        mask = (seg == gids).astype(jnp.float32)

        @pl.when(i == 0)
        def _():
            acc[...] = jnp.zeros_like(acc)
            cnt[...] = jnp.zeros_like(cnt)

        acc[...] += jnp.dot(mask, h2, preferred_element_type=jnp.float32)
        cnt[...] += jnp.broadcast_to(
            jnp.sum(mask, axis=1, keepdims=True), (G, H))

        @pl.when(i == nblk - 1)
        def _():
            o_ref[...] = jnp.maximum(
                acc[...] / jnp.maximum(cnt[...], 1.0), 0.0)

    return pl.pallas_call(
        body,
        grid=(nblk,),
        in_specs=[
            pl.BlockSpec((NC, _BLK, H), lambda i: (0, i, 0)),
            pl.BlockSpec((_BLK, H), lambda i: (i, 0)),
            pl.BlockSpec((H, H), lambda i: (0, 0)),
            pl.BlockSpec((1, H), lambda i: (0, 0)),
            pl.BlockSpec((H, H), lambda i: (0, 0)),
            pl.BlockSpec((1, 1, _BLK), lambda i: (i, 0, 0)),
        ],
        out_specs=pl.BlockSpec((G, H), lambda i: (0, 0)),
        out_shape=jax.ShapeDtypeStruct((G, H), jnp.float32),
        scratch_shapes=[
            pltpu.VMEM((G, H), jnp.float32),
            pltpu.VMEM((G, H), jnp.float32),
        ],
    )(p, h, W_rel2, b_rel2, W_root2, batch3)


def kernel(x, edge_index, batch, W_rel1, b_rel1, W_root1,
           W_rel2, b_rel2, W_root2):
    src_flat = edge_index[0]
    dst = edge_index[1].reshape(NW, NCHUNK, CHUNK)
    zeros = jnp.zeros((NPAD, H), jnp.float32)
    batch3 = batch.reshape(N // _BLK, 1, _BLK)

    p1 = _sc_aggregate(x, src_flat, dst, zeros)
    hmid = _tc_mid(p1, x, W_rel1, b_rel1.reshape(1, H), W_root1)
    p2 = _sc_aggregate(hmid, src_flat, dst, zeros)
    return _tc_pool(p2, hmid, W_rel2, b_rel2.reshape(1, H),
                    W_root2, batch3)
